# Initial kernel scaffold; baseline (speedup 1.0000x reference)
import jax
import jax.numpy as jnp
from jax.experimental import pallas as pl


def kernel(atomic_numbers, distances, i, j, index_i, index_j, index_k, index_ji, index_kj, batch, emb, W_filt, b_filt, W_c2, b_c2, g_c2, bb_c2, g_c22, bb_c22, W_c3, b_c3, g_c3, bb_c3, g_c32, bb_c32, W_c1, b_c1, g_c1, bb_c1, g_fn, bb_fn, W_out, b_out):
    raise NotImplementedError("write your pallas kernel here")


# speedup vs baseline: 1.0834x; 1.0834x over previous
import functools

import jax
import jax.numpy as jnp
from jax import lax
from jax.experimental import pallas as pl
from jax.experimental.pallas import tpu as pltpu

_N, _E, _T, _S = 10000, 160000, 160000, 16
_DN, _DE, _STEPS, _NSP, _L = 256, 128, 64, 95, 2
_GSTART, _GSTOP = 0.0, 5.0
_BR = 640
_BN = 1000


def _ln(x, g, b):
    m = jnp.mean(x, axis=-1, keepdims=True)
    v = jnp.mean((x - m) ** 2, axis=-1, keepdims=True)
    return (x - m) / jnp.sqrt(v + 1e-5) * g + b


def _rows(d, br=_BR):
    return pl.BlockSpec((br, d), lambda r: (r, 0))


def _resident(shape):
    return pl.BlockSpec(shape, lambda r: tuple(0 for _ in shape))


def _edge0(distances, W_filt, b_filt):
    step = (_GSTOP - _GSTART) / (_STEPS - 1)
    coeff = -0.5 / step ** 2

    def body(d_ref, w_ref, b_ref, o_ref):
        d = d_ref[:, :]
        off = _GSTART + step * lax.broadcasted_iota(
            jnp.int32, (1, _STEPS), 1).astype(jnp.float32)
        gf = jnp.exp(coeff * (d - off) ** 2)
        o_ref[:, :] = (jnp.dot(gf, w_ref[:, :], preferred_element_type=jnp.float32, precision=lax.Precision.HIGHEST)
                       + b_ref[:, :])

    return pl.pallas_call(
        body,
        grid=(_E // _BR,),
        in_specs=[_rows(1), _resident((_STEPS, _DE)), _resident((1, _DE))],
        out_specs=_rows(_DE),
        out_shape=jax.ShapeDtypeStruct((_E, _DE), jnp.float32),
    )(distances.reshape(_E, 1), W_filt, b_filt.reshape(1, _DE))


def _node0(atomic_numbers, emb):
    embp = jnp.pad(emb, ((0, 128 - _NSP), (0, 0)))

    def body(a_ref, e_ref, o_ref):
        oh = (a_ref[:, :] == lax.broadcasted_iota(jnp.int32, (1, 128), 1)
              ).astype(jnp.float32)
        o_ref[:, :] = jnp.dot(oh, e_ref[:, :], preferred_element_type=jnp.float32, precision=lax.Precision.HIGHEST)

    return pl.pallas_call(
        body,
        grid=(_N // _BN,),
        in_specs=[_rows(1, _BN), _resident((128, _DN))],
        out_specs=_rows(_DN, _BN),
        out_shape=jax.ShapeDtypeStruct((_N, _DN), jnp.float32),
    )(atomic_numbers.reshape(_N, 1).astype(jnp.int32), embp)


def _c2(ni, nj, W, b, g, bb, g2, bb2):
    def body(ni_ref, nj_ref, w_ref, b_ref, g_ref, bb_ref, g2_ref, bb2_ref, o_ref):
        x = ni_ref[:, :] * nj_ref[:, :]
        h = jnp.dot(x, w_ref[:, :], preferred_element_type=jnp.float32, precision=lax.Precision.HIGHEST) + b_ref[:, :]
        h = _ln(h, g_ref[:, :], bb_ref[:, :])
        e = jax.nn.sigmoid(h[:, :_DE]) * jnp.tanh(h[:, _DE:])
        o_ref[:, :] = _ln(e, g2_ref[:, :], bb2_ref[:, :])

    return pl.pallas_call(
        body,
        grid=(_E // _BR,),
        in_specs=[_rows(_DN), _rows(_DN), _resident((_DN, 2 * _DE)),
                  _resident((1, 2 * _DE)), _resident((1, 2 * _DE)),
                  _resident((1, 2 * _DE)), _resident((1, _DE)), _resident((1, _DE))],
        out_specs=_rows(_DE),
        out_shape=jax.ShapeDtypeStruct((_E, _DE), jnp.float32),
    )(ni, nj, W, b.reshape(1, -1), g.reshape(1, -1), bb.reshape(1, -1),
      g2.reshape(1, -1), bb2.reshape(1, -1))


def _c3(ni, nj, nk, eji, ekj, W, b, g, bb):
    W1, W2, W3 = W[:_DN], W[_DN:2 * _DN], W[2 * _DN:3 * _DN]
    W4, W5 = W[3 * _DN:3 * _DN + _DE], W[3 * _DN + _DE:]

    def body(ni_ref, nj_ref, nk_ref, eji_ref, ekj_ref,
             w1_ref, w2_ref, w3_ref, w4_ref, w5_ref, b_ref, g_ref, bb_ref, o_ref):
        h = jnp.dot(ni_ref[:, :], w1_ref[:, :], preferred_element_type=jnp.float32, precision=lax.Precision.HIGHEST)
        h += jnp.dot(nj_ref[:, :], w2_ref[:, :], preferred_element_type=jnp.float32, precision=lax.Precision.HIGHEST)
        h += jnp.dot(nk_ref[:, :], w3_ref[:, :], preferred_element_type=jnp.float32, precision=lax.Precision.HIGHEST)
        h += jnp.dot(eji_ref[:, :], w4_ref[:, :], preferred_element_type=jnp.float32, precision=lax.Precision.HIGHEST)
        h += jnp.dot(ekj_ref[:, :], w5_ref[:, :], preferred_element_type=jnp.float32, precision=lax.Precision.HIGHEST)
        h = _ln(h + b_ref[:, :], g_ref[:, :], bb_ref[:, :])
        o_ref[:, :] = jax.nn.sigmoid(h[:, :_DE]) * jnp.tanh(h[:, _DE:])

    return pl.pallas_call(
        body,
        grid=(_T // _BR,),
        in_specs=[_rows(_DN), _rows(_DN), _rows(_DN), _rows(_DE), _rows(_DE),
                  _resident((_DN, 2 * _DE)), _resident((_DN, 2 * _DE)),
                  _resident((_DN, 2 * _DE)), _resident((_DE, 2 * _DE)),
                  _resident((_DE, 2 * _DE)), _resident((1, 2 * _DE)),
                  _resident((1, 2 * _DE)), _resident((1, 2 * _DE))],
        out_specs=_rows(_DE),
        out_shape=jax.ShapeDtypeStruct((_T, _DE), jnp.float32),
    )(ni, nj, nk, eji, ekj, W1, W2, W3, W4, W5,
      b.reshape(1, -1), g.reshape(1, -1), bb.reshape(1, -1))


def _edge_update(edge, c2e, c3raw, g32, bb32):
    def body(e_ref, c2_ref, c3_ref, g_ref, bb_ref, o_ref):
        c3e = _ln(c3_ref[:, :], g_ref[:, :], bb_ref[:, :])
        o_ref[:, :] = jnp.tanh(e_ref[:, :] + c2_ref[:, :] + c3e)

    return pl.pallas_call(
        body,
        grid=(_E // _BR,),
        in_specs=[_rows(_DE), _rows(_DE), _rows(_DE),
                  _resident((1, _DE)), _resident((1, _DE))],
        out_specs=_rows(_DE),
        out_shape=jax.ShapeDtypeStruct((_E, _DE), jnp.float32),
    )(edge, c2e, c3raw, g32.reshape(1, -1), bb32.reshape(1, -1))


def _c1(ni, edge, W, b, g, bb):
    Wa, Wb = W[:_DN], W[_DN:]

    def body(ni_ref, e_ref, wa_ref, wb_ref, b_ref, g_ref, bb_ref, o_ref):
        h = jnp.dot(ni_ref[:, :], wa_ref[:, :], preferred_element_type=jnp.float32, precision=lax.Precision.HIGHEST)
        h += jnp.dot(e_ref[:, :], wb_ref[:, :], preferred_element_type=jnp.float32, precision=lax.Precision.HIGHEST)
        h = _ln(h + b_ref[:, :], g_ref[:, :], bb_ref[:, :])
        o_ref[:, :] = jax.nn.sigmoid(h[:, :_DN]) * jnp.tanh(h[:, _DN:])

    return pl.pallas_call(
        body,
        grid=(_E // _BR,),
        in_specs=[_rows(_DN), _rows(_DE), _resident((_DN, 2 * _DN)),
                  _resident((_DE, 2 * _DN)), _resident((1, 2 * _DN)),
                  _resident((1, 2 * _DN)), _resident((1, 2 * _DN))],
        out_specs=_rows(_DN),
        out_shape=jax.ShapeDtypeStruct((_E, _DN), jnp.float32),
    )(ni, edge, Wa, Wb, b.reshape(1, -1), g.reshape(1, -1), bb.reshape(1, -1))


def _node_update(node, agg, g, bb):
    def body(n_ref, a_ref, g_ref, bb_ref, o_ref):
        o_ref[:, :] = jnp.tanh(_ln(n_ref[:, :] + a_ref[:, :],
                                   g_ref[:, :], bb_ref[:, :]))

    return pl.pallas_call(
        body,
        grid=(_N // _BN,),
        in_specs=[_rows(_DN, _BN), _rows(_DN, _BN),
                  _resident((1, _DN)), _resident((1, _DN))],
        out_specs=_rows(_DN, _BN),
        out_shape=jax.ShapeDtypeStruct((_N, _DN), jnp.float32),
    )(node, agg, g.reshape(1, -1), bb.reshape(1, -1))


def _readout(node, batch, W_out, b_out):
    nblk = _N // _BN

    def body(n_ref, b_ref, w_ref, bo_ref, o_ref, acc):
        r = pl.program_id(0)
        oh = (b_ref[:, :] == lax.broadcasted_iota(jnp.int32, (1, _S), 1)
              ).astype(jnp.float32)
        c = lax.dot_general(oh, n_ref[:, :], (((0,), (0,)), ((), ())),
                            preferred_element_type=jnp.float32, precision=lax.Precision.HIGHEST)

        @pl.when(r == 0)
        def _():
            acc[:, :] = c

        @pl.when(r > 0)
        def _():
            acc[:, :] = acc[:, :] + c

        @pl.when(r == nblk - 1)
        def _():
            o_ref[:, :] = (jnp.dot(acc[:, :], w_ref[:, :],
                                   preferred_element_type=jnp.float32, precision=lax.Precision.HIGHEST)
                           + bo_ref[:, :])

    return pl.pallas_call(
        body,
        grid=(nblk,),
        in_specs=[_rows(_DN, _BN), _rows(1, _BN),
                  _resident((_DN, 1)), _resident((1, 1))],
        out_specs=_resident((_S, 1)),
        out_shape=jax.ShapeDtypeStruct((_S, 1), jnp.float32),
        scratch_shapes=[pltpu.VMEM((_S, _DN), jnp.float32)],
    )(node, batch.reshape(_N, 1).astype(jnp.int32), W_out, b_out.reshape(1, 1))


def kernel(atomic_numbers, distances, i, j, index_i, index_j, index_k,
           index_ji, index_kj, batch, emb, W_filt, b_filt, W_c2, b_c2, g_c2,
           bb_c2, g_c22, bb_c22, W_c3, b_c3, g_c3, bb_c3, g_c32, bb_c32,
           W_c1, b_c1, g_c1, bb_c1, g_fn, bb_fn, W_out, b_out):
    i = i.astype(jnp.int32)
    j = j.astype(jnp.int32)
    index_i = index_i.astype(jnp.int32)
    index_j = index_j.astype(jnp.int32)
    index_k = index_k.astype(jnp.int32)
    index_ji = index_ji.astype(jnp.int32)
    index_kj = index_kj.astype(jnp.int32)

    edge = _edge0(distances, W_filt, b_filt)
    node = _node0(atomic_numbers, emb)
    for l in range(_L):
        ni = jnp.take(node, i, axis=0)
        nj = jnp.take(node, j, axis=0)
        c2e = _c2(ni, nj, W_c2[l], b_c2[l], g_c2[l], bb_c2[l], g_c22[l], bb_c22[l])
        t_ni = jnp.take(node, index_i, axis=0)
        t_nj = jnp.take(node, index_j, axis=0)
        t_nk = jnp.take(node, index_k, axis=0)
        t_eji = jnp.take(edge, index_ji, axis=0)
        t_ekj = jnp.take(edge, index_kj, axis=0)
        c3m = _c3(t_ni, t_nj, t_nk, t_eji, t_ekj, W_c3[l], b_c3[l], g_c3[l], bb_c3[l])
        c3raw = jax.ops.segment_sum(c3m, index_ji, num_segments=_E)
        edge = _edge_update(edge, c2e, c3raw, g_c32[l], bb_c32[l])
        c1m = _c1(ni, edge, W_c1[l], b_c1[l], g_c1[l], bb_c1[l])
        agg = jax.ops.segment_sum(c1m, i, num_segments=_N)
        node = _node_update(node, agg, g_fn[l], bb_fn[l])
    return _readout(node, batch, W_out, b_out)


# speedup vs baseline: 1.2232x; 1.1291x over previous
import functools

import jax
import jax.numpy as jnp
from jax import lax
from jax.experimental import pallas as pl
from jax.experimental.pallas import tpu as pltpu
from jax.experimental.pallas import tpu_sc as plsc

_N, _E, _T, _S = 10000, 160000, 160000, 16
_DN, _DE, _STEPS, _NSP, _L = 256, 128, 64, 95, 2
_GSTART, _GSTOP = 0.0, 5.0
_BR = 640
_BN = 1000


_NC, _NS = 2, 16
_NW = _NC * _NS
_CH = 128
_BP = 163840


def _sc_gather(tasks):
    per_w = _BP // _NW
    nch = per_w // _CH
    dims = sorted({int(t.shape[1]) for t, _ in tasks}, reverse=True)
    out_type = [jax.ShapeDtypeStruct((_BP, int(t.shape[1])), jnp.float32)
                for t, _ in tasks]
    scratch = [pltpu.VMEM((per_w,), jnp.int32)]
    for d in dims:
        scratch += [pltpu.VMEM((_CH, d), jnp.float32),
                    pltpu.VMEM((_CH, d), jnp.float32)]
    scratch += [pltpu.SemaphoreType.DMA, pltpu.SemaphoreType.DMA]
    mesh = plsc.VectorSubcoreMesh(core_axis_name="c", subcore_axis_name="s")
    nt = len(tasks)

    def body(*refs):
        tbls = refs[0:2 * nt:2]
        idxs = refs[1:2 * nt:2]
        outs = refs[2 * nt:3 * nt]
        idx_v = refs[3 * nt]
        bufs = {d: (refs[3 * nt + 1 + 2 * k], refs[3 * nt + 2 + 2 * k])
                for k, d in enumerate(dims)}
        sem0, sem1 = refs[-2], refs[-1]
        wid = lax.axis_index("s") * _NC + lax.axis_index("c")
        base = wid * per_w
        for t in range(nt):
            d = int(tasks[t][0].shape[1])
            b0, b1 = bufs[d]
            pltpu.sync_copy(idxs[t].at[pl.ds(base, per_w)], idx_v)

            def grp(g, _, tbl=tbls[t], out=outs[t], b0=b0, b1=b1):
                a = g * 2
                cpa = pltpu.async_copy(
                    tbl.at[idx_v.at[pl.ds(a * _CH, _CH)]], b0, sem0)
                cpb = pltpu.async_copy(
                    tbl.at[idx_v.at[pl.ds((a + 1) * _CH, _CH)]], b1, sem1)
                cpa.wait()
                pltpu.sync_copy(b0, out.at[pl.ds(base + a * _CH, _CH)])
                cpb.wait()
                pltpu.sync_copy(b1, out.at[pl.ds(base + (a + 1) * _CH, _CH)])
                return 0

            lax.fori_loop(0, nch // 2, grp, 0)

    args = []
    for t, idx in tasks:
        args += [t, idx]
    f = pl.kernel(body, out_type=out_type, mesh=mesh, scratch_types=scratch)
    outs = f(*args)
    return outs if nt > 1 else (outs,)


def _ln(x, g, b):
    m = jnp.mean(x, axis=-1, keepdims=True)
    v = jnp.mean((x - m) ** 2, axis=-1, keepdims=True)
    return (x - m) / jnp.sqrt(v + 1e-5) * g + b


def _rows(d, br=_BR):
    return pl.BlockSpec((br, d), lambda r: (r, 0))


def _resident(shape):
    return pl.BlockSpec(shape, lambda r: tuple(0 for _ in shape))


def _edge0(distances, W_filt, b_filt):
    step = (_GSTOP - _GSTART) / (_STEPS - 1)
    coeff = -0.5 / step ** 2

    def body(d_ref, w_ref, b_ref, o_ref):
        d = d_ref[:, :]
        off = _GSTART + step * lax.broadcasted_iota(
            jnp.int32, (1, _STEPS), 1).astype(jnp.float32)
        gf = jnp.exp(coeff * (d - off) ** 2)
        o_ref[:, :] = (jnp.dot(gf, w_ref[:, :], preferred_element_type=jnp.float32, precision=lax.Precision.HIGHEST)
                       + b_ref[:, :])

    return pl.pallas_call(
        body,
        grid=(_E // _BR,),
        in_specs=[_rows(1), _resident((_STEPS, _DE)), _resident((1, _DE))],
        out_specs=_rows(_DE),
        out_shape=jax.ShapeDtypeStruct((_E, _DE), jnp.float32),
    )(distances.reshape(_E, 1), W_filt, b_filt.reshape(1, _DE))


def _node0(atomic_numbers, emb):
    embp = jnp.pad(emb, ((0, 128 - _NSP), (0, 0)))

    def body(a_ref, e_ref, o_ref):
        oh = (a_ref[:, :] == lax.broadcasted_iota(jnp.int32, (1, 128), 1)
              ).astype(jnp.float32)
        o_ref[:, :] = jnp.dot(oh, e_ref[:, :], preferred_element_type=jnp.float32, precision=lax.Precision.HIGHEST)

    return pl.pallas_call(
        body,
        grid=(_N // _BN,),
        in_specs=[_rows(1, _BN), _resident((128, _DN))],
        out_specs=_rows(_DN, _BN),
        out_shape=jax.ShapeDtypeStruct((_N, _DN), jnp.float32),
    )(atomic_numbers.reshape(_N, 1).astype(jnp.int32), embp)


def _c2(ni, nj, W, b, g, bb, g2, bb2):
    def body(ni_ref, nj_ref, w_ref, b_ref, g_ref, bb_ref, g2_ref, bb2_ref, o_ref):
        x = ni_ref[:, :] * nj_ref[:, :]
        h = jnp.dot(x, w_ref[:, :], preferred_element_type=jnp.float32, precision=lax.Precision.HIGHEST) + b_ref[:, :]
        h = _ln(h, g_ref[:, :], bb_ref[:, :])
        e = jax.nn.sigmoid(h[:, :_DE]) * jnp.tanh(h[:, _DE:])
        o_ref[:, :] = _ln(e, g2_ref[:, :], bb2_ref[:, :])

    return pl.pallas_call(
        body,
        grid=(_E // _BR,),
        in_specs=[_rows(_DN), _rows(_DN), _resident((_DN, 2 * _DE)),
                  _resident((1, 2 * _DE)), _resident((1, 2 * _DE)),
                  _resident((1, 2 * _DE)), _resident((1, _DE)), _resident((1, _DE))],
        out_specs=_rows(_DE),
        out_shape=jax.ShapeDtypeStruct((_E, _DE), jnp.float32),
    )(ni, nj, W, b.reshape(1, -1), g.reshape(1, -1), bb.reshape(1, -1),
      g2.reshape(1, -1), bb2.reshape(1, -1))


def _c3(ni, nj, nk, eji, ekj, W, b, g, bb):
    W1, W2, W3 = W[:_DN], W[_DN:2 * _DN], W[2 * _DN:3 * _DN]
    W4, W5 = W[3 * _DN:3 * _DN + _DE], W[3 * _DN + _DE:]

    def body(ni_ref, nj_ref, nk_ref, eji_ref, ekj_ref,
             w1_ref, w2_ref, w3_ref, w4_ref, w5_ref, b_ref, g_ref, bb_ref, o_ref):
        h = jnp.dot(ni_ref[:, :], w1_ref[:, :], preferred_element_type=jnp.float32, precision=lax.Precision.HIGHEST)
        h += jnp.dot(nj_ref[:, :], w2_ref[:, :], preferred_element_type=jnp.float32, precision=lax.Precision.HIGHEST)
        h += jnp.dot(nk_ref[:, :], w3_ref[:, :], preferred_element_type=jnp.float32, precision=lax.Precision.HIGHEST)
        h += jnp.dot(eji_ref[:, :], w4_ref[:, :], preferred_element_type=jnp.float32, precision=lax.Precision.HIGHEST)
        h += jnp.dot(ekj_ref[:, :], w5_ref[:, :], preferred_element_type=jnp.float32, precision=lax.Precision.HIGHEST)
        h = _ln(h + b_ref[:, :], g_ref[:, :], bb_ref[:, :])
        o_ref[:, :] = jax.nn.sigmoid(h[:, :_DE]) * jnp.tanh(h[:, _DE:])

    return pl.pallas_call(
        body,
        grid=(_T // _BR,),
        in_specs=[_rows(_DN), _rows(_DN), _rows(_DN), _rows(_DE), _rows(_DE),
                  _resident((_DN, 2 * _DE)), _resident((_DN, 2 * _DE)),
                  _resident((_DN, 2 * _DE)), _resident((_DE, 2 * _DE)),
                  _resident((_DE, 2 * _DE)), _resident((1, 2 * _DE)),
                  _resident((1, 2 * _DE)), _resident((1, 2 * _DE))],
        out_specs=_rows(_DE),
        out_shape=jax.ShapeDtypeStruct((_T, _DE), jnp.float32),
    )(ni, nj, nk, eji, ekj, W1, W2, W3, W4, W5,
      b.reshape(1, -1), g.reshape(1, -1), bb.reshape(1, -1))


def _edge_update(edge, c2e, c3raw, g32, bb32):
    def body(e_ref, c2_ref, c3_ref, g_ref, bb_ref, o_ref):
        c3e = _ln(c3_ref[:, :], g_ref[:, :], bb_ref[:, :])
        o_ref[:, :] = jnp.tanh(e_ref[:, :] + c2_ref[:, :] + c3e)

    return pl.pallas_call(
        body,
        grid=(_E // _BR,),
        in_specs=[_rows(_DE), _rows(_DE), _rows(_DE),
                  _resident((1, _DE)), _resident((1, _DE))],
        out_specs=_rows(_DE),
        out_shape=jax.ShapeDtypeStruct((_E, _DE), jnp.float32),
    )(edge, c2e, c3raw, g32.reshape(1, -1), bb32.reshape(1, -1))


def _c1(ni, edge, W, b, g, bb):
    Wa, Wb = W[:_DN], W[_DN:]

    def body(ni_ref, e_ref, wa_ref, wb_ref, b_ref, g_ref, bb_ref, o_ref):
        h = jnp.dot(ni_ref[:, :], wa_ref[:, :], preferred_element_type=jnp.float32, precision=lax.Precision.HIGHEST)
        h += jnp.dot(e_ref[:, :], wb_ref[:, :], preferred_element_type=jnp.float32, precision=lax.Precision.HIGHEST)
        h = _ln(h + b_ref[:, :], g_ref[:, :], bb_ref[:, :])
        o_ref[:, :] = jax.nn.sigmoid(h[:, :_DN]) * jnp.tanh(h[:, _DN:])

    return pl.pallas_call(
        body,
        grid=(_E // _BR,),
        in_specs=[_rows(_DN), _rows(_DE), _resident((_DN, 2 * _DN)),
                  _resident((_DE, 2 * _DN)), _resident((1, 2 * _DN)),
                  _resident((1, 2 * _DN)), _resident((1, 2 * _DN))],
        out_specs=_rows(_DN),
        out_shape=jax.ShapeDtypeStruct((_E, _DN), jnp.float32),
    )(ni, edge, Wa, Wb, b.reshape(1, -1), g.reshape(1, -1), bb.reshape(1, -1))


def _node_update(node, agg, g, bb):
    def body(n_ref, a_ref, g_ref, bb_ref, o_ref):
        o_ref[:, :] = jnp.tanh(_ln(n_ref[:, :] + a_ref[:, :],
                                   g_ref[:, :], bb_ref[:, :]))

    return pl.pallas_call(
        body,
        grid=(_N // _BN,),
        in_specs=[_rows(_DN, _BN), _rows(_DN, _BN),
                  _resident((1, _DN)), _resident((1, _DN))],
        out_specs=_rows(_DN, _BN),
        out_shape=jax.ShapeDtypeStruct((_N, _DN), jnp.float32),
    )(node, agg, g.reshape(1, -1), bb.reshape(1, -1))


def _readout(node, batch, W_out, b_out):
    nblk = _N // _BN

    def body(n_ref, b_ref, w_ref, bo_ref, o_ref, acc):
        r = pl.program_id(0)
        oh = (b_ref[:, :] == lax.broadcasted_iota(jnp.int32, (1, _S), 1)
              ).astype(jnp.float32)
        c = lax.dot_general(oh, n_ref[:, :], (((0,), (0,)), ((), ())),
                            preferred_element_type=jnp.float32, precision=lax.Precision.HIGHEST)

        @pl.when(r == 0)
        def _():
            acc[:, :] = c

        @pl.when(r > 0)
        def _():
            acc[:, :] = acc[:, :] + c

        @pl.when(r == nblk - 1)
        def _():
            o_ref[:, :] = (jnp.dot(acc[:, :], w_ref[:, :],
                                   preferred_element_type=jnp.float32, precision=lax.Precision.HIGHEST)
                           + bo_ref[:, :])

    return pl.pallas_call(
        body,
        grid=(nblk,),
        in_specs=[_rows(_DN, _BN), _rows(1, _BN),
                  _resident((_DN, 1)), _resident((1, 1))],
        out_specs=_resident((_S, 1)),
        out_shape=jax.ShapeDtypeStruct((_S, 1), jnp.float32),
        scratch_shapes=[pltpu.VMEM((_S, _DN), jnp.float32)],
    )(node, batch.reshape(_N, 1).astype(jnp.int32), W_out, b_out.reshape(1, 1))


def kernel(atomic_numbers, distances, i, j, index_i, index_j, index_k,
           index_ji, index_kj, batch, emb, W_filt, b_filt, W_c2, b_c2, g_c2,
           bb_c2, g_c22, bb_c22, W_c3, b_c3, g_c3, bb_c3, g_c32, bb_c32,
           W_c1, b_c1, g_c1, bb_c1, g_fn, bb_fn, W_out, b_out):
    i = i.astype(jnp.int32)
    j = j.astype(jnp.int32)
    index_i = index_i.astype(jnp.int32)
    index_j = index_j.astype(jnp.int32)
    index_k = index_k.astype(jnp.int32)
    index_ji = index_ji.astype(jnp.int32)
    index_kj = index_kj.astype(jnp.int32)

    pad = (0, _BP - _E)
    ip, jp = jnp.pad(i, pad), jnp.pad(j, pad)
    tip, tjp, tkp = jnp.pad(index_i, pad), jnp.pad(index_j, pad), jnp.pad(index_k, pad)
    jip, kjp = jnp.pad(index_ji, pad), jnp.pad(index_kj, pad)

    edge = _edge0(distances, W_filt, b_filt)
    node = _node0(atomic_numbers, emb)
    for l in range(_L):
        ni, nj = _sc_gather([(node, ip), (node, jp)])
        c2e = _c2(ni, nj, W_c2[l], b_c2[l], g_c2[l], bb_c2[l], g_c22[l], bb_c22[l])
        t_ni, t_nj, t_nk, t_eji, t_ekj = _sc_gather(
            [(node, tip), (node, tjp), (node, tkp), (edge, jip), (edge, kjp)])
        c3m = _c3(t_ni, t_nj, t_nk, t_eji, t_ekj, W_c3[l], b_c3[l], g_c3[l], bb_c3[l])
        c3raw = jax.ops.segment_sum(c3m, index_ji, num_segments=_E)
        edge = _edge_update(edge, c2e, c3raw, g_c32[l], bb_c32[l])
        c1m = _c1(ni, edge, W_c1[l], b_c1[l], g_c1[l], bb_c1[l])
        agg = jax.ops.segment_sum(c1m, i, num_segments=_N)
        node = _node_update(node, agg, g_fn[l], bb_fn[l])
    return _readout(node, batch, W_out, b_out)


# speedup vs baseline: 1.4684x; 1.2004x over previous
import functools

import jax
import jax.numpy as jnp
from jax import lax
from jax.experimental import pallas as pl
from jax.experimental.pallas import tpu as pltpu
from jax.experimental.pallas import tpu_sc as plsc

_N, _E, _T, _S = 10000, 160000, 160000, 16
_DN, _DE, _STEPS, _NSP, _L = 256, 128, 64, 95, 2
_GSTART, _GSTOP = 0.0, 5.0
_BR = 640
_BN = 1000


_NC, _NS = 2, 16
_NW = _NC * _NS
_CH = 128
_BP = 163840


def _sc_gather(tasks):
    per_w = _BP // _NW
    nch = per_w // _CH
    dims = sorted({int(t.shape[1]) for t, _ in tasks}, reverse=True)
    out_type = [jax.ShapeDtypeStruct((_BP, int(t.shape[1])), jnp.float32)
                for t, _ in tasks]
    scratch = [pltpu.VMEM((per_w,), jnp.int32)]
    for d in dims:
        scratch += [pltpu.VMEM((_CH, d), jnp.float32),
                    pltpu.VMEM((_CH, d), jnp.float32)]
    scratch += [pltpu.SemaphoreType.DMA, pltpu.SemaphoreType.DMA]
    mesh = plsc.VectorSubcoreMesh(core_axis_name="c", subcore_axis_name="s")
    nt = len(tasks)

    def body(*refs):
        tbls = refs[0:2 * nt:2]
        idxs = refs[1:2 * nt:2]
        outs = refs[2 * nt:3 * nt]
        idx_v = refs[3 * nt]
        bufs = {d: (refs[3 * nt + 1 + 2 * k], refs[3 * nt + 2 + 2 * k])
                for k, d in enumerate(dims)}
        sem0, sem1 = refs[-2], refs[-1]
        wid = lax.axis_index("s") * _NC + lax.axis_index("c")
        base = wid * per_w
        for t in range(nt):
            d = int(tasks[t][0].shape[1])
            b0, b1 = bufs[d]
            pltpu.sync_copy(idxs[t].at[pl.ds(base, per_w)], idx_v)

            def grp(g, _, tbl=tbls[t], out=outs[t], b0=b0, b1=b1):
                a = g * 2
                cpa = pltpu.async_copy(
                    tbl.at[idx_v.at[pl.ds(a * _CH, _CH)]], b0, sem0)
                cpb = pltpu.async_copy(
                    tbl.at[idx_v.at[pl.ds((a + 1) * _CH, _CH)]], b1, sem1)
                cpa.wait()
                pltpu.sync_copy(b0, out.at[pl.ds(base + a * _CH, _CH)])
                cpb.wait()
                pltpu.sync_copy(b1, out.at[pl.ds(base + (a + 1) * _CH, _CH)])
                return 0

            lax.fori_loop(0, nch // 2, grp, 0)

    args = []
    for t, idx in tasks:
        args += [t, idx]
    f = pl.kernel(body, out_type=out_type, mesh=mesh, scratch_types=scratch)
    outs = f(*args)
    return outs if nt > 1 else (outs,)


_NACC = 10240


def _sc_scatter_node(d0, d1, idx2d, zeros):
    per_t = _BP // _NS
    nch = per_t // _CH
    mesh = plsc.VectorSubcoreMesh(core_axis_name="c", subcore_axis_name="s")
    out_type = [jax.ShapeDtypeStruct((_N, 128), jnp.float32)] * 2
    scratch = [
        pltpu.VMEM_SHARED((_NACC, 128), jnp.float32),
        pltpu.VMEM((_CH, 128), jnp.float32),
        pltpu.VMEM((_CH, 128), jnp.float32),
        pltpu.VMEM((nch, _CH), jnp.int32),
        pltpu.SemaphoreType.DMA,
        pltpu.SemaphoreType.DMA,
    ]
    zrows = _NACC // _NS

    def body(d0_ref, d1_ref, idx_ref, z_ref, o0_ref, o1_ref,
             acc, bufa, bufb, idx_v, sema, semb):
        c = lax.axis_index("c")
        s = lax.axis_index("s")
        pltpu.sync_copy(z_ref, acc.at[pl.ds(s * zrows, zrows)])
        pltpu.sync_copy(idx_ref.at[pl.ds(s * nch, nch)], idx_v)
        plsc.subcore_barrier()

        def scatter_rows(dref):
            def chunk(g, _):
                a = g * 2
                cpa = pltpu.async_copy(
                    dref.at[pl.ds(s * per_t + a * _CH, _CH)], bufa, sema)
                cpb = pltpu.async_copy(
                    dref.at[pl.ds(s * per_t + (a + 1) * _CH, _CH)], bufb, semb)
                cpa.wait()
                pltpu.sync_copy(bufa, acc.at[idx_v.at[a]], add=True)
                cpb.wait()
                pltpu.sync_copy(bufb, acc.at[idx_v.at[a + 1]], add=True)
                return 0
            lax.fori_loop(0, nch // 2, chunk, 0)

        @pl.when(c == 0)
        def _():
            scatter_rows(d0_ref)

        @pl.when(c == 1)
        def _():
            scatter_rows(d1_ref)

        plsc.subcore_barrier()
        rpt = 624

        def copy_out(o_ref):
            pltpu.sync_copy(acc.at[pl.ds(s * rpt, rpt)], o_ref.at[pl.ds(s * rpt, rpt)])

            @pl.when(s == 0)
            def _():
                pltpu.sync_copy(acc.at[pl.ds(_NS * rpt, _N - _NS * rpt)],
                                o_ref.at[pl.ds(_NS * rpt, _N - _NS * rpt)])

        @pl.when(c == 0)
        def _():
            copy_out(o0_ref)

        @pl.when(c == 1)
        def _():
            copy_out(o1_ref)

    f = pl.kernel(body, out_type=out_type, mesh=mesh, scratch_types=scratch)
    return f(d0, d1, idx2d, zeros)


def _ln(x, g, b):
    m = jnp.mean(x, axis=-1, keepdims=True)
    v = jnp.mean((x - m) ** 2, axis=-1, keepdims=True)
    return (x - m) / jnp.sqrt(v + 1e-5) * g + b


def _rows(d, br=_BR):
    return pl.BlockSpec((br, d), lambda r: (r, 0))


def _resident(shape):
    return pl.BlockSpec(shape, lambda r: tuple(0 for _ in shape))


def _edge0(distances, W_filt, b_filt):
    step = (_GSTOP - _GSTART) / (_STEPS - 1)
    coeff = -0.5 / step ** 2

    def body(d_ref, w_ref, b_ref, o_ref):
        d = d_ref[:, :]
        off = _GSTART + step * lax.broadcasted_iota(
            jnp.int32, (1, _STEPS), 1).astype(jnp.float32)
        gf = jnp.exp(coeff * (d - off) ** 2)
        o_ref[:, :] = (jnp.dot(gf, w_ref[:, :], preferred_element_type=jnp.float32, precision=lax.Precision.DEFAULT)
                       + b_ref[:, :])

    return pl.pallas_call(
        body,
        grid=(_BP // _BR,),
        in_specs=[_rows(1), _resident((_STEPS, _DE)), _resident((1, _DE))],
        out_specs=_rows(_DE),
        out_shape=jax.ShapeDtypeStruct((_BP, _DE), jnp.float32),
    )(distances.reshape(_BP, 1), W_filt, b_filt.reshape(1, _DE))


def _node0(atomic_numbers, emb):
    embp = jnp.pad(emb, ((0, 128 - _NSP), (0, 0)))

    def body(a_ref, e_ref, o_ref):
        oh = (a_ref[:, :] == lax.broadcasted_iota(jnp.int32, (1, 128), 1)
              ).astype(jnp.float32)
        o_ref[:, :] = jnp.dot(oh, e_ref[:, :], preferred_element_type=jnp.float32, precision=lax.Precision.HIGHEST)

    return pl.pallas_call(
        body,
        grid=(_N // _BN,),
        in_specs=[_rows(1, _BN), _resident((128, _DN))],
        out_specs=_rows(_DN, _BN),
        out_shape=jax.ShapeDtypeStruct((_N, _DN), jnp.float32),
    )(atomic_numbers.reshape(_N, 1).astype(jnp.int32), embp)


def _c2(ni, nj, W, b, g, bb, g2, bb2):
    def body(ni_ref, nj_ref, w_ref, b_ref, g_ref, bb_ref, g2_ref, bb2_ref, o_ref):
        x = ni_ref[:, :] * nj_ref[:, :]
        h = jnp.dot(x, w_ref[:, :], preferred_element_type=jnp.float32, precision=lax.Precision.DEFAULT) + b_ref[:, :]
        h = _ln(h, g_ref[:, :], bb_ref[:, :])
        e = jax.nn.sigmoid(h[:, :_DE]) * jnp.tanh(h[:, _DE:])
        o_ref[:, :] = _ln(e, g2_ref[:, :], bb2_ref[:, :])

    return pl.pallas_call(
        body,
        grid=(_BP // _BR,),
        in_specs=[_rows(_DN), _rows(_DN), _resident((_DN, 2 * _DE)),
                  _resident((1, 2 * _DE)), _resident((1, 2 * _DE)),
                  _resident((1, 2 * _DE)), _resident((1, _DE)), _resident((1, _DE))],
        out_specs=_rows(_DE),
        out_shape=jax.ShapeDtypeStruct((_BP, _DE), jnp.float32),
    )(ni, nj, W, b.reshape(1, -1), g.reshape(1, -1), bb.reshape(1, -1),
      g2.reshape(1, -1), bb2.reshape(1, -1))


def _c3(ni, nj, nk, eji, ekj, W, b, g, bb):
    W1, W2, W3 = W[:_DN], W[_DN:2 * _DN], W[2 * _DN:3 * _DN]
    W4, W5 = W[3 * _DN:3 * _DN + _DE], W[3 * _DN + _DE:]

    def body(ni_ref, nj_ref, nk_ref, eji_ref, ekj_ref,
             w1_ref, w2_ref, w3_ref, w4_ref, w5_ref, b_ref, g_ref, bb_ref, o_ref):
        h = jnp.dot(ni_ref[:, :], w1_ref[:, :], preferred_element_type=jnp.float32, precision=lax.Precision.DEFAULT)
        h += jnp.dot(nj_ref[:, :], w2_ref[:, :], preferred_element_type=jnp.float32, precision=lax.Precision.DEFAULT)
        h += jnp.dot(nk_ref[:, :], w3_ref[:, :], preferred_element_type=jnp.float32, precision=lax.Precision.DEFAULT)
        h += jnp.dot(eji_ref[:, :], w4_ref[:, :], preferred_element_type=jnp.float32, precision=lax.Precision.DEFAULT)
        h += jnp.dot(ekj_ref[:, :], w5_ref[:, :], preferred_element_type=jnp.float32, precision=lax.Precision.DEFAULT)
        h = _ln(h + b_ref[:, :], g_ref[:, :], bb_ref[:, :])
        o_ref[:, :] = jax.nn.sigmoid(h[:, :_DE]) * jnp.tanh(h[:, _DE:])

    return pl.pallas_call(
        body,
        grid=(_BP // _BR,),
        in_specs=[_rows(_DN), _rows(_DN), _rows(_DN), _rows(_DE), _rows(_DE),
                  _resident((_DN, 2 * _DE)), _resident((_DN, 2 * _DE)),
                  _resident((_DN, 2 * _DE)), _resident((_DE, 2 * _DE)),
                  _resident((_DE, 2 * _DE)), _resident((1, 2 * _DE)),
                  _resident((1, 2 * _DE)), _resident((1, 2 * _DE))],
        out_specs=_rows(_DE),
        out_shape=jax.ShapeDtypeStruct((_BP, _DE), jnp.float32),
    )(ni, nj, nk, eji, ekj, W1, W2, W3, W4, W5,
      b.reshape(1, -1), g.reshape(1, -1), bb.reshape(1, -1))


def _edge_update(edge, c2e, c3raw, g32, bb32):
    def body(e_ref, c2_ref, c3_ref, g_ref, bb_ref, o_ref):
        c3e = _ln(c3_ref[:, :], g_ref[:, :], bb_ref[:, :])
        o_ref[:, :] = jnp.tanh(e_ref[:, :] + c2_ref[:, :] + c3e)

    return pl.pallas_call(
        body,
        grid=(_BP // _BR,),
        in_specs=[_rows(_DE), _rows(_DE), _rows(_DE),
                  _resident((1, _DE)), _resident((1, _DE))],
        out_specs=_rows(_DE),
        out_shape=jax.ShapeDtypeStruct((_BP, _DE), jnp.float32),
    )(edge, c2e, c3raw, g32.reshape(1, -1), bb32.reshape(1, -1))


def _c1(ni, edge, W, b, g, bb):
    Wa, Wb = W[:_DN], W[_DN:]

    def body(ni_ref, e_ref, wa_ref, wb_ref, b_ref, g_ref, bb_ref, o0_ref, o1_ref):
        h = jnp.dot(ni_ref[:, :], wa_ref[:, :], preferred_element_type=jnp.float32, precision=lax.Precision.DEFAULT)
        h += jnp.dot(e_ref[:, :], wb_ref[:, :], preferred_element_type=jnp.float32, precision=lax.Precision.DEFAULT)
        h = _ln(h + b_ref[:, :], g_ref[:, :], bb_ref[:, :])
        m = jax.nn.sigmoid(h[:, :_DN]) * jnp.tanh(h[:, _DN:])
        o0_ref[:, :] = m[:, :_DE]
        o1_ref[:, :] = m[:, _DE:]

    return pl.pallas_call(
        body,
        grid=(_BP // _BR,),
        in_specs=[_rows(_DN), _rows(_DE), _resident((_DN, 2 * _DN)),
                  _resident((_DE, 2 * _DN)), _resident((1, 2 * _DN)),
                  _resident((1, 2 * _DN)), _resident((1, 2 * _DN))],
        out_specs=[_rows(_DE), _rows(_DE)],
        out_shape=[jax.ShapeDtypeStruct((_BP, _DE), jnp.float32),
                   jax.ShapeDtypeStruct((_BP, _DE), jnp.float32)],
    )(ni, edge, Wa, Wb, b.reshape(1, -1), g.reshape(1, -1), bb.reshape(1, -1))


def _node_update(node, agg0, agg1, g, bb):
    def body(n_ref, a0_ref, a1_ref, g_ref, bb_ref, o_ref):
        agg = jnp.concatenate([a0_ref[:, :], a1_ref[:, :]], axis=1)
        o_ref[:, :] = jnp.tanh(_ln(n_ref[:, :] + agg,
                                   g_ref[:, :], bb_ref[:, :]))

    return pl.pallas_call(
        body,
        grid=(_N // _BN,),
        in_specs=[_rows(_DN, _BN), _rows(_DE, _BN), _rows(_DE, _BN),
                  _resident((1, _DN)), _resident((1, _DN))],
        out_specs=_rows(_DN, _BN),
        out_shape=jax.ShapeDtypeStruct((_N, _DN), jnp.float32),
    )(node, agg0, agg1, g.reshape(1, -1), bb.reshape(1, -1))


def _readout(node, batch, W_out, b_out):
    nblk = _N // _BN

    def body(n_ref, b_ref, w_ref, bo_ref, o_ref, acc):
        r = pl.program_id(0)
        oh = (b_ref[:, :] == lax.broadcasted_iota(jnp.int32, (1, _S), 1)
              ).astype(jnp.float32)
        c = lax.dot_general(oh, n_ref[:, :], (((0,), (0,)), ((), ())),
                            preferred_element_type=jnp.float32, precision=lax.Precision.HIGHEST)

        @pl.when(r == 0)
        def _():
            acc[:, :] = c

        @pl.when(r > 0)
        def _():
            acc[:, :] = acc[:, :] + c

        @pl.when(r == nblk - 1)
        def _():
            o_ref[:, :] = (jnp.dot(acc[:, :], w_ref[:, :],
                                   preferred_element_type=jnp.float32, precision=lax.Precision.DEFAULT)
                           + bo_ref[:, :])

    return pl.pallas_call(
        body,
        grid=(nblk,),
        in_specs=[_rows(_DN, _BN), _rows(1, _BN),
                  _resident((_DN, 1)), _resident((1, 1))],
        out_specs=_resident((_S, 1)),
        out_shape=jax.ShapeDtypeStruct((_S, 1), jnp.float32),
        scratch_shapes=[pltpu.VMEM((_S, _DN), jnp.float32)],
    )(node, batch.reshape(_N, 1).astype(jnp.int32), W_out, b_out.reshape(1, 1))


def kernel(atomic_numbers, distances, i, j, index_i, index_j, index_k,
           index_ji, index_kj, batch, emb, W_filt, b_filt, W_c2, b_c2, g_c2,
           bb_c2, g_c22, bb_c22, W_c3, b_c3, g_c3, bb_c3, g_c32, bb_c32,
           W_c1, b_c1, g_c1, bb_c1, g_fn, bb_fn, W_out, b_out):
    i = i.astype(jnp.int32)
    j = j.astype(jnp.int32)
    index_i = index_i.astype(jnp.int32)
    index_j = index_j.astype(jnp.int32)
    index_k = index_k.astype(jnp.int32)
    index_ji = index_ji.astype(jnp.int32)
    index_kj = index_kj.astype(jnp.int32)

    pad = (0, _BP - _E)
    ip, jp = jnp.pad(i, pad), jnp.pad(j, pad)
    tip, tjp, tkp = jnp.pad(index_i, pad), jnp.pad(index_j, pad), jnp.pad(index_k, pad)
    jip, kjp = jnp.pad(index_ji, pad), jnp.pad(index_kj, pad)
    i_scat = jnp.pad(i, pad, constant_values=_N).reshape(_BP // _CH, _CH)
    zeros = jnp.zeros((_NACC // _NS, 128), jnp.float32)

    edge = _edge0(jnp.pad(distances, pad), W_filt, b_filt)
    node = _node0(atomic_numbers, emb)
    for l in range(_L):
        ni, nj = _sc_gather([(node, ip), (node, jp)])
        c2e = _c2(ni, nj, W_c2[l], b_c2[l], g_c2[l], bb_c2[l], g_c22[l], bb_c22[l])
        t_ni, t_nj, t_nk, t_eji, t_ekj = _sc_gather(
            [(node, tip), (node, tjp), (node, tkp), (edge, jip), (edge, kjp)])
        c3m = _c3(t_ni, t_nj, t_nk, t_eji, t_ekj, W_c3[l], b_c3[l], g_c3[l], bb_c3[l])
        c3raw = jnp.pad(jax.ops.segment_sum(c3m[:_E], index_ji, num_segments=_E),
                        (pad, (0, 0)))
        edge = _edge_update(edge, c2e, c3raw, g_c32[l], bb_c32[l])
        c10, c11 = _c1(ni, edge, W_c1[l], b_c1[l], g_c1[l], bb_c1[l])
        agg0, agg1 = _sc_scatter_node(c10, c11, i_scat, zeros)
        node = _node_update(node, agg0, agg1, g_fn[l], bb_fn[l])
    return _readout(node, batch, W_out, b_out)
